# baseline (speedup 1.0000x reference)
import functools

import jax
import jax.numpy as jnp
from jax import lax
from jax.experimental import pallas as pl
from jax.experimental.pallas import tpu as pltpu
from jax.experimental.pallas import tpu_sc as plsc

RES = 16
B = 64
N = 100000
F = RES ** 3
C = 40

NC = 2
NS = 16
L = 16
NW = NC * NS
BPW = B // NW
CH = 4000
CH3 = CH * 3
NCH = N // CH
VPC = CH // L

_mesh = plsc.VectorSubcoreMesh(
    core_axis_name="c", subcore_axis_name="s", num_cores=NC, num_subcores=NS
)


@functools.partial(
    pl.kernel,
    out_type=jax.ShapeDtypeStruct((B * F,), jnp.float32),
    mesh=_mesh,
    scratch_types=[
        pltpu.VMEM((CH3,), jnp.float32),
        pltpu.VMEM((L, F), jnp.float32),
        pltpu.VMEM((F,), jnp.float32),
    ],
    compiler_params=pltpu.CompilerParams(
        use_tc_tiling_on_sc=False, needs_layout_passes=False
    ),
)
def _hist_sc(x_ref, out_ref, buf, hists, hist):
    cid = lax.axis_index("c")
    sid = lax.axis_index("s")
    wid = sid * NC + cid

    lane = lax.iota(jnp.int32, 16)
    iota3 = lane * 3
    ones = jnp.ones((16,), jnp.float32)
    zeros = jnp.zeros((16,), jnp.float32)

    for l in range(L):
        @pl.loop(0, F // 16)
        def _zero(j):
            hists[l, pl.ds(j * 16, 16)] = zeros

    for bi in range(BPW):
        b = wid * BPW + bi

        def chunk_minmax(k, carry):
            pltpu.sync_copy(
                x_ref.at[pl.ds(pl.multiple_of(b * (N * 3) + k * CH3, 8), CH3)],
                buf,
            )

            def inner(i, c):
                mnx, mxx, mny, mxy, mnz, mxz = c
                idx = i * 48 + iota3
                xs = plsc.load_gather(buf, [idx])
                ys = plsc.load_gather(buf, [idx + 1])
                zs = plsc.load_gather(buf, [idx + 2])
                return (
                    jnp.minimum(mnx, xs), jnp.maximum(mxx, xs),
                    jnp.minimum(mny, ys), jnp.maximum(mxy, ys),
                    jnp.minimum(mnz, zs), jnp.maximum(mxz, zs),
                )

            return lax.fori_loop(0, VPC, inner, carry)

        big = jnp.full((16,), jnp.inf, jnp.float32)
        mnx, mxx, mny, mxy, mnz, mxz = lax.fori_loop(
            0, NCH, chunk_minmax, (big, -big, big, -big, big, -big)
        )

        def _prep(mn, mx):
            mn_s = jnp.full((16,), jnp.min(mn))
            rng = jnp.full((16,), jnp.max(mx)) - mn_s
            rng = jnp.where(rng <= 0.0, ones, rng)
            scale = jnp.full((16,), float(RES), jnp.float32) / rng
            return mn_s, scale

        mnxv, sclx = _prep(mnx, mxx)
        mnyv, scly = _prep(mny, mxy)
        mnzv, sclz = _prep(mnz, mxz)
        fifteen = jnp.full((16,), 15, jnp.int32)

        def chunk_bin(k, carry):
            pltpu.sync_copy(
                x_ref.at[pl.ds(pl.multiple_of(b * (N * 3) + k * CH3, 8), CH3)],
                buf,
            )

            def inner(i, c):
                idx = i * 48 + iota3
                xs = plsc.load_gather(buf, [idx])
                ys = plsc.load_gather(buf, [idx + 1])
                zs = plsc.load_gather(buf, [idx + 2])
                ix = jnp.minimum(((xs - mnxv) * sclx).astype(jnp.int32), fifteen)
                iy = jnp.minimum(((ys - mnyv) * scly).astype(jnp.int32), fifteen)
                iz = jnp.minimum(((zs - mnzv) * sclz).astype(jnp.int32), fifteen)
                flat = (ix << 8) | (iy << 4) | iz
                plsc.addupdate_scatter(hists, [lane, flat], ones)
                return c

            return lax.fori_loop(0, VPC, inner, 0)

        lax.fori_loop(0, NCH, chunk_bin, 0)

        @pl.loop(0, F // 16)
        def _reduce(j):
            col = pl.ds(j * 16, 16)
            acc = hists[0, col]
            hists[0, col] = zeros
            for l in range(1, L):
                acc = acc + hists[l, col]
                hists[l, col] = zeros
            hist[col] = acc

        pltpu.sync_copy(hist, out_ref.at[pl.ds(pl.multiple_of(b * F, 8), F)])


def _mm_body(c_ref, w_ref, b_ref, o_ref):
    c = c_ref[...] * (1.0 / float(N))
    o_ref[...] = (
        lax.dot_general(
            c, w_ref[...], (((1,), (1,)), ((), ())),
            preferred_element_type=jnp.float32,
            precision=lax.Precision.HIGHEST,
        )
        + b_ref[...]
    )


def kernel(x, W, b):
    counts = _hist_sc(x.reshape(B * N * 3)).reshape(B, F)
    logits = pl.pallas_call(
        _mm_body,
        out_shape=jax.ShapeDtypeStruct((B, C), jnp.float32),
    )(counts, W, b.reshape(1, C))
    return logits


# speedup vs baseline: 52.6214x; 52.6214x over previous
import functools

import jax
import jax.numpy as jnp
from jax import lax
from jax.experimental import pallas as pl
from jax.experimental.pallas import tpu as pltpu
from jax.experimental.pallas import tpu_sc as plsc

RES = 16
B = 64
N = 100000
F = RES ** 3
C = 40

NC = 2
NS = 16
NQ = 4
NG = 8
NT = 782
CW = 17
CHC = CW * 128
NCHUNK = NT // CW
LAST = NCHUNK - 1
NV_FULL = CHC // 16
NV_LAST = (100000 - LAST * CHC) // 16

_mesh = plsc.VectorSubcoreMesh(
    core_axis_name="c", subcore_axis_name="s", num_cores=NC, num_subcores=NS
)


@functools.partial(
    pl.kernel,
    out_type=jax.ShapeDtypeStruct((NQ, B, F), jnp.float32),
    mesh=_mesh,
    scratch_types=[
        pltpu.VMEM((3, 8, CHC), jnp.float32),
        pltpu.VMEM((8, F), jnp.float32),
        pltpu.VMEM((8, 768), jnp.float32),
        pltpu.VMEM((4, 8, 768), jnp.float32),
        pltpu.VMEM((8, 768), jnp.float32),
        pltpu.VMEM_SHARED((NS, 8, 768), jnp.float32),
        pltpu.SemaphoreType.DMA,
    ],
    compiler_params=pltpu.CompilerParams(needs_layout_passes=False),
)
def _hist_sc(x_ref, out_ref, bufs, hists, acc, mmbuf, prm, shared_mm, sem):
    cid = lax.axis_index("c")
    sid = lax.axis_index("s")
    wid = cid * NS + sid
    g = wid // NQ
    q = wid % NQ
    nchunks = jnp.where(q < 2, 12, 11)

    ones = jnp.ones((16,), jnp.float32)
    zeros = jnp.zeros((16,), jnp.float32)
    posinf = jnp.full((16,), jnp.inf, jnp.float32)
    neginf = jnp.full((16,), -jnp.inf, jnp.float32)
    fifteen = jnp.full((16,), 15, jnp.int32)

    for r in range(8):
        @pl.loop(0, F // 16)
        def _zero(j):
            hists[r, pl.ds(j * 16, 16)] = zeros
    for d in range(3):
        @pl.loop(0, 768 // 16)
        def _init(j):
            acc[2 * d, pl.ds(j * 16, 16)] = posinf
            acc[2 * d + 1, pl.ds(j * 16, 16)] = neginf

    def fetch_chunk(c):
        col = pl.multiple_of(c * CHC, 128)
        row = pl.multiple_of(g * 8, 8)
        hs = [
            pltpu.async_copy(
                x_ref.at[d, pl.ds(row, 8), pl.ds(col, CHC)], bufs.at[d], sem
            )
            for d in range(3)
        ]
        for h in hs:
            h.wait()

    def chunk_minmax(j, carry):
        c = q + NQ * j
        fetch_chunk(c)
        nv = jnp.where(c == LAST, NV_LAST, NV_FULL)
        for d in range(3):
            for r in range(8):
                col = pl.ds(r * 16, 16)
                mn0 = acc[2 * d, col]
                mx0 = acc[2 * d + 1, col]

                def inner(v, mm, d=d, r=r):
                    mn, mx = mm
                    vals = bufs[d, r, pl.ds(v * 16, 16)]
                    return jnp.minimum(mn, vals), jnp.maximum(mx, vals)

                mm = lax.fori_loop(0, NV_LAST, inner, (mn0, mx0), unroll=4)
                mn1, mx1 = lax.fori_loop(NV_LAST, nv, inner, mm)
                acc[2 * d, col] = mn1
                acc[2 * d + 1, col] = mx1
        return carry

    lax.fori_loop(0, nchunks, chunk_minmax, 0)

    pltpu.sync_copy(acc, shared_mm.at[sid])
    plsc.subcore_barrier()
    peer0 = pl.multiple_of((sid // NQ) * NQ, 4)
    pltpu.sync_copy(shared_mm.at[pl.ds(peer0, 4)], mmbuf)

    for d in range(3):
        for r in range(8):
            col = pl.ds(r * 16, 16)
            mn = jnp.minimum(
                jnp.minimum(mmbuf[0, 2 * d, col], mmbuf[1, 2 * d, col]),
                jnp.minimum(mmbuf[2, 2 * d, col], mmbuf[3, 2 * d, col]),
            )
            mx = jnp.maximum(
                jnp.maximum(mmbuf[0, 2 * d + 1, col], mmbuf[1, 2 * d + 1, col]),
                jnp.maximum(mmbuf[2, 2 * d + 1, col], mmbuf[3, 2 * d + 1, col]),
            )
            mn_s = jnp.full((16,), jnp.min(mn))
            rng = jnp.full((16,), jnp.max(mx)) - mn_s
            rng = jnp.where(rng <= 0.0, ones, rng)
            prm[2 * d, col] = mn_s
            prm[2 * d + 1, col] = jnp.full((16,), float(RES), jnp.float32) / rng

    def chunk_bin(j, carry):
        c = q + NQ * j
        fetch_chunk(c)
        nv = jnp.where(c == LAST, NV_LAST, NV_FULL)
        for r in range(8):
            col = pl.ds(r * 16, 16)
            mnx, sclx = prm[0, col], prm[1, col]
            mny, scly = prm[2, col], prm[3, col]
            mnz, sclz = prm[4, col], prm[5, col]
            row = jnp.full((16,), r, jnp.int32)

            def inner(v, cc, r=r, mnx=mnx, sclx=sclx, mny=mny, scly=scly,
                      mnz=mnz, sclz=sclz, row=row):
                s = pl.ds(v * 16, 16)
                xs = bufs[0, r, s]
                ys = bufs[1, r, s]
                zs = bufs[2, r, s]
                ix = jnp.minimum(((xs - mnx) * sclx).astype(jnp.int32), fifteen)
                iy = jnp.minimum(((ys - mny) * scly).astype(jnp.int32), fifteen)
                iz = jnp.minimum(((zs - mnz) * sclz).astype(jnp.int32), fifteen)
                flat = (ix << 8) | (iy << 4) | iz
                plsc.addupdate_scatter(hists, [row, flat], ones)
                return cc

            lax.fori_loop(0, NV_LAST, inner, 0, unroll=4)
            lax.fori_loop(NV_LAST, nv, inner, 0)
        return carry

    lax.fori_loop(0, nchunks, chunk_bin, 0)

    pltpu.sync_copy(hists, out_ref.at[q, pl.ds(pl.multiple_of(g * 8, 8), 8), :])


def _mm_body(p_ref, w_ref, b_ref, o_ref):
    c = (p_ref[0] + p_ref[1] + p_ref[2] + p_ref[3]) * (1.0 / float(N))
    o_ref[...] = (
        lax.dot_general(
            c, w_ref[...], (((1,), (1,)), ((), ())),
            preferred_element_type=jnp.float32,
            precision=lax.Precision.HIGHEST,
        )
        + b_ref[...]
    )


def kernel(x, W, b):
    xt = jnp.transpose(x, (2, 0, 1))
    partials = _hist_sc(xt)
    logits = pl.pallas_call(
        _mm_body,
        out_shape=jax.ShapeDtypeStruct((B, C), jnp.float32),
    )(partials, W, b.reshape(1, C))
    return logits


# speedup vs baseline: 84.8932x; 1.6133x over previous
import functools

import jax
import jax.numpy as jnp
from jax import lax
from jax.experimental import pallas as pl
from jax.experimental.pallas import tpu as pltpu
from jax.experimental.pallas import tpu_sc as plsc

RES = 16
B = 64
N = 100000
F = RES ** 3
C = 40

NC = 2
NS = 16
NQ = 4
NT = 782
CW = 17
CHC = CW * 128
NCHUNK = NT // CW
LAST = NCHUNK - 1
NV_FULL = CHC // 16
NV_LAST = (100000 - LAST * CHC) // 16
N4_FULL = NV_FULL // 4
N4_LAST = NV_LAST // 4

_mesh = plsc.VectorSubcoreMesh(
    core_axis_name="c", subcore_axis_name="s", num_cores=NC, num_subcores=NS
)


@functools.partial(
    pl.kernel,
    out_type=jax.ShapeDtypeStruct((NQ, B, F), jnp.float32),
    mesh=_mesh,
    scratch_types=[
        pltpu.VMEM((3, 8, CHC), jnp.float32),
        pltpu.VMEM((8 * F,), jnp.float32),
        pltpu.VMEM((8, F), jnp.float32),
        pltpu.VMEM((8, 128), jnp.float32),
        pltpu.VMEM((4, 8, 128), jnp.float32),
        pltpu.VMEM((8, 128), jnp.float32),
        pltpu.VMEM_SHARED((NS, 8, 128), jnp.float32),
        pltpu.SemaphoreType.DMA,
    ],
    compiler_params=pltpu.CompilerParams(needs_layout_passes=False),
)
def _hist_sc(x_ref, out_ref, bufs, hist1, hist2, acc, mmbuf, prm, shared_mm,
             sem):
    cid = lax.axis_index("c")
    sid = lax.axis_index("s")
    wid = cid * NS + sid
    g = wid // NQ
    q = wid % NQ
    nchunks = jnp.where(q < 2, 12, 11)

    ones = jnp.ones((16,), jnp.float32)
    zeros = jnp.zeros((16,), jnp.float32)
    posinf = jnp.full((16,), jnp.inf, jnp.float32)
    neginf = jnp.full((16,), -jnp.inf, jnp.float32)
    f15 = jnp.full((16,), 15.0, jnp.float32)

    @pl.loop(0, 8 * F // 16)
    def _zero(j):
        hist1[pl.ds(j * 16, 16)] = zeros

    for d in range(3):
        for r in range(8):
            acc[2 * d, pl.ds(r * 16, 16)] = posinf
            acc[2 * d + 1, pl.ds(r * 16, 16)] = neginf

    def fetch_chunk(c):
        col = pl.multiple_of(c * CHC, 128)
        row = pl.multiple_of(g * 8, 8)
        hs = [
            pltpu.async_copy(
                x_ref.at[d, pl.ds(row, 8), pl.ds(col, CHC)], bufs.at[d], sem
            )
            for d in range(3)
        ]
        for h in hs:
            h.wait()

    def chunk_minmax(j, carry):
        c = q + NQ * j
        fetch_chunk(c)
        last = c == LAST
        n4 = jnp.where(last, N4_LAST, N4_FULL)
        nv = jnp.where(last, NV_LAST, NV_FULL)
        for d in range(3):
            for r in range(8):
                col = pl.ds(r * 16, 16)

                def quad(v4, mm, d=d, r=r):
                    a0, a1, a2, a3, b0, b1, b2, b3 = mm
                    base = v4 * 64
                    v0 = bufs[d, r, pl.ds(base, 16)]
                    v1 = bufs[d, r, pl.ds(base + 16, 16)]
                    v2 = bufs[d, r, pl.ds(base + 32, 16)]
                    v3 = bufs[d, r, pl.ds(base + 48, 16)]
                    return (
                        jnp.minimum(a0, v0), jnp.minimum(a1, v1),
                        jnp.minimum(a2, v2), jnp.minimum(a3, v3),
                        jnp.maximum(b0, v0), jnp.maximum(b1, v1),
                        jnp.maximum(b2, v2), jnp.maximum(b3, v3),
                    )

                def single(v, mm, d=d, r=r):
                    a0, a1, a2, a3, b0, b1, b2, b3 = mm
                    vals = bufs[d, r, pl.ds(v * 16, 16)]
                    return (jnp.minimum(a0, vals), a1, a2, a3,
                            jnp.maximum(b0, vals), b1, b2, b3)

                init = (acc[2 * d, col], posinf, posinf, posinf,
                        acc[2 * d + 1, col], neginf, neginf, neginf)
                mm = lax.fori_loop(0, N4_LAST, quad, init, unroll=2)
                mm = lax.fori_loop(N4_LAST, n4, quad, mm)
                mm = lax.fori_loop(4 * n4, nv, single, mm)
                a0, a1, a2, a3, b0, b1, b2, b3 = mm
                acc[2 * d, col] = jnp.minimum(
                    jnp.minimum(a0, a1), jnp.minimum(a2, a3))
                acc[2 * d + 1, col] = jnp.maximum(
                    jnp.maximum(b0, b1), jnp.maximum(b2, b3))
        return carry

    lax.fori_loop(0, nchunks, chunk_minmax, 0)

    pltpu.sync_copy(acc, shared_mm.at[sid])
    plsc.subcore_barrier()
    peer0 = pl.multiple_of((sid // NQ) * NQ, 4)
    pltpu.sync_copy(shared_mm.at[pl.ds(peer0, 4)], mmbuf)

    for d in range(3):
        for r in range(8):
            col = pl.ds(r * 16, 16)
            mn = jnp.minimum(
                jnp.minimum(mmbuf[0, 2 * d, col], mmbuf[1, 2 * d, col]),
                jnp.minimum(mmbuf[2, 2 * d, col], mmbuf[3, 2 * d, col]),
            )
            mx = jnp.maximum(
                jnp.maximum(mmbuf[0, 2 * d + 1, col], mmbuf[1, 2 * d + 1, col]),
                jnp.maximum(mmbuf[2, 2 * d + 1, col], mmbuf[3, 2 * d + 1, col]),
            )
            mn_s = jnp.full((16,), jnp.min(mn))
            rng = jnp.full((16,), jnp.max(mx)) - mn_s
            rng = jnp.where(rng <= 0.0, ones, rng)
            prm[2 * d, col] = mn_s
            prm[2 * d + 1, col] = jnp.full((16,), float(RES), jnp.float32) / rng

    def chunk_bin(j, carry):
        c = q + NQ * j
        fetch_chunk(c)
        last = c == LAST
        n4 = jnp.where(last, N4_LAST, N4_FULL)
        nv = jnp.where(last, NV_LAST, NV_FULL)
        for r in range(8):
            col = pl.ds(r * 16, 16)
            mnx, sclx = prm[0, col], prm[1, col]
            mny, scly = prm[2, col], prm[3, col]
            mnz, sclz = prm[4, col], prm[5, col]
            hrow = hist1.at[pl.ds(r * F, F)]

            def binvec(xs, ys, zs, mnx=mnx, sclx=sclx, mny=mny, scly=scly,
                       mnz=mnz, sclz=sclz):
                ix = jnp.minimum((xs - mnx) * sclx, f15).astype(jnp.int32)
                iy = jnp.minimum((ys - mny) * scly, f15).astype(jnp.int32)
                iz = jnp.minimum((zs - mnz) * sclz, f15).astype(jnp.int32)
                return (ix << 8) | (iy << 4) | iz

            def quad(v4, cc, r=r, hrow=hrow, binvec=binvec):
                base = v4 * 64
                xs = [bufs[0, r, pl.ds(base + 16 * i, 16)] for i in range(4)]
                ys = [bufs[1, r, pl.ds(base + 16 * i, 16)] for i in range(4)]
                zs = [bufs[2, r, pl.ds(base + 16 * i, 16)] for i in range(4)]
                flats = [binvec(xs[i], ys[i], zs[i]) for i in range(4)]
                for i in range(4):
                    plsc.addupdate_scatter(hrow, [flats[i]], ones)
                return cc

            def single(v, cc, r=r, hrow=hrow, binvec=binvec):
                s = pl.ds(v * 16, 16)
                flat = binvec(bufs[0, r, s], bufs[1, r, s], bufs[2, r, s])
                plsc.addupdate_scatter(hrow, [flat], ones)
                return cc

            lax.fori_loop(0, N4_LAST, quad, 0, unroll=2)
            lax.fori_loop(N4_LAST, n4, quad, 0)
            lax.fori_loop(4 * n4, nv, single, 0)
        return carry

    lax.fori_loop(0, nchunks, chunk_bin, 0)

    for r in range(8):
        @pl.loop(0, F // 16)
        def _conv(jj, r=r):
            hist2[r, pl.ds(jj * 16, 16)] = hist1[pl.ds(r * F + jj * 16, 16)]

    pltpu.sync_copy(hist2, out_ref.at[q, pl.ds(pl.multiple_of(g * 8, 8), 8), :])


def _mm_body(p_ref, w_ref, b_ref, o_ref):
    c = (p_ref[0] + p_ref[1] + p_ref[2] + p_ref[3]) * (1.0 / float(N))
    o_ref[...] = (
        lax.dot_general(
            c, w_ref[...], (((1,), (1,)), ((), ())),
            preferred_element_type=jnp.float32,
            precision=lax.Precision.HIGHEST,
        )
        + b_ref[...]
    )


def kernel(x, W, b):
    xt = jnp.transpose(x, (2, 0, 1))
    partials = _hist_sc(xt)
    logits = pl.pallas_call(
        _mm_body,
        out_shape=jax.ShapeDtypeStruct((B, C), jnp.float32),
    )(partials, W, b.reshape(1, C))
    return logits


# speedup vs baseline: 113.4015x; 1.3358x over previous
import functools

import jax
import jax.numpy as jnp
from jax import lax
from jax.experimental import pallas as pl
from jax.experimental.pallas import tpu as pltpu
from jax.experimental.pallas import tpu_sc as plsc

RES = 16
B = 64
N = 100000
F = RES ** 3
C = 40

NC = 2
NS = 16
NQ = 4
NT = 782
CW = 8
CHC = CW * 128
NCHUNK = 98
LAST = NCHUNK - 1
TAIL_COL = NT * 128 - CHC
TAIL_V0 = (97 * CHC - TAIL_COL) // 16
TAIL_V1 = (100000 - TAIL_COL) // 16
NV = CHC // 16

_mesh = plsc.VectorSubcoreMesh(
    core_axis_name="c", subcore_axis_name="s", num_cores=NC, num_subcores=NS
)


@functools.partial(
    pl.kernel,
    out_type=jax.ShapeDtypeStruct((NQ, B, F), jnp.float32),
    mesh=_mesh,
    scratch_types=[
        pltpu.VMEM((2, 3, 8, CHC), jnp.float32),
        pltpu.VMEM((8 * F,), jnp.float32),
        pltpu.VMEM((8, F), jnp.float32),
        pltpu.VMEM((8, 128), jnp.float32),
        pltpu.VMEM((4, 8, 128), jnp.float32),
        pltpu.VMEM((8, 128), jnp.float32),
        pltpu.VMEM_SHARED((NS, 8, 128), jnp.float32),
        pltpu.SemaphoreType.DMA,
        pltpu.SemaphoreType.DMA,
    ],
    compiler_params=pltpu.CompilerParams(needs_layout_passes=False),
)
def _hist_sc(x_ref, out_ref, bufs, hist1, hist2, acc, mmbuf, prm, shared_mm,
             semA, semB):
    cid = lax.axis_index("c")
    sid = lax.axis_index("s")
    wid = cid * NS + sid
    g = wid // NQ
    q = wid % NQ
    nq = jnp.where(q < 2, 25, 24)
    row = pl.multiple_of(g * 8, 8)

    ones = jnp.ones((16,), jnp.float32)
    zeros = jnp.zeros((16,), jnp.float32)
    posinf = jnp.full((16,), jnp.inf, jnp.float32)
    neginf = jnp.full((16,), -jnp.inf, jnp.float32)
    f15 = jnp.full((16,), 15.0, jnp.float32)

    @pl.loop(0, 8 * F // 16)
    def _zero(j):
        hist1[pl.ds(j * 16, 16)] = zeros

    for d in range(3):
        for r in range(8):
            acc[2 * d, pl.ds(r * 16, 16)] = posinf
            acc[2 * d + 1, pl.ds(r * 16, 16)] = neginf

    def start_fetch(c, slot, sem):
        col = pl.multiple_of(
            jnp.where(c == LAST, TAIL_COL, c * CHC).astype(jnp.int32), 128
        )
        for d in range(3):
            pltpu.async_copy(
                x_ref.at[d, pl.ds(row, 8), pl.ds(col, CHC)],
                bufs.at[slot, d], sem
            )

    def wait_fetch(slot, sem):
        for d in range(3):
            pltpu.make_async_copy(
                x_ref.at[d, pl.ds(row, 8), pl.ds(0, CHC)],
                bufs.at[slot, d], sem
            ).wait()

    def run_pass(process):
        start_fetch(q, 0, semA)

        def step(k, carry):
            start_fetch(q + NQ * (2 * k + 1), 1, semB)
            wait_fetch(0, semA)
            process(0, False)

            @pl.when(2 * k + 2 < nq)
            def _():
                start_fetch(q + NQ * (2 * k + 2), 0, semA)

            wait_fetch(1, semB)
            process(1, False)
            return carry

        lax.fori_loop(0, nq // 2, step, 0)

        @pl.when(nq % 2 == 1)
        def _():
            wait_fetch(0, semA)

            @pl.when(q == 1)
            def _():
                process(0, True)

            @pl.when(q != 1)
            def _():
                process(0, False)

    def mm_process(slot, tail):
        v0 = TAIL_V0 if tail else 0
        n4 = (TAIL_V1 - TAIL_V0) // 4 if tail else NV // 4
        v1 = TAIL_V1 if tail else NV
        for d in range(3):
            @pl.loop(0, 8)
            def _leg(r, d=d):
                col = pl.ds(r * 16, 16)

                def quad(j4, mm, d=d, r=r):
                    a0, a1, a2, a3, b0, b1, b2, b3 = mm
                    base = (v0 + 4 * j4) * 16
                    w0 = bufs[slot, d, r, pl.ds(base, 16)]
                    w1 = bufs[slot, d, r, pl.ds(base + 16, 16)]
                    w2 = bufs[slot, d, r, pl.ds(base + 32, 16)]
                    w3 = bufs[slot, d, r, pl.ds(base + 48, 16)]
                    return (
                        jnp.minimum(a0, w0), jnp.minimum(a1, w1),
                        jnp.minimum(a2, w2), jnp.minimum(a3, w3),
                        jnp.maximum(b0, w0), jnp.maximum(b1, w1),
                        jnp.maximum(b2, w2), jnp.maximum(b3, w3),
                    )

                def single(v, mm, d=d, r=r):
                    a0, a1, a2, a3, b0, b1, b2, b3 = mm
                    vals = bufs[slot, d, r, pl.ds(v * 16, 16)]
                    return (jnp.minimum(a0, vals), a1, a2, a3,
                            jnp.maximum(b0, vals), b1, b2, b3)

                init = (acc[2 * d, col], posinf, posinf, posinf,
                        acc[2 * d + 1, col], neginf, neginf, neginf)
                mm = lax.fori_loop(0, n4, quad, init, unroll=2)
                mm = lax.fori_loop(v0 + 4 * n4, v1, single, mm)
                a0, a1, a2, a3, b0, b1, b2, b3 = mm
                acc[2 * d, col] = jnp.minimum(
                    jnp.minimum(a0, a1), jnp.minimum(a2, a3))
                acc[2 * d + 1, col] = jnp.maximum(
                    jnp.maximum(b0, b1), jnp.maximum(b2, b3))

    run_pass(mm_process)

    pltpu.sync_copy(acc, shared_mm.at[sid])
    plsc.subcore_barrier()
    peer0 = pl.multiple_of((sid // NQ) * NQ, 4)
    pltpu.sync_copy(shared_mm.at[pl.ds(peer0, 4)], mmbuf)

    for d in range(3):
        for r in range(8):
            col = pl.ds(r * 16, 16)
            mn = jnp.minimum(
                jnp.minimum(mmbuf[0, 2 * d, col], mmbuf[1, 2 * d, col]),
                jnp.minimum(mmbuf[2, 2 * d, col], mmbuf[3, 2 * d, col]),
            )
            mx = jnp.maximum(
                jnp.maximum(mmbuf[0, 2 * d + 1, col], mmbuf[1, 2 * d + 1, col]),
                jnp.maximum(mmbuf[2, 2 * d + 1, col], mmbuf[3, 2 * d + 1, col]),
            )
            mn_s = jnp.full((16,), jnp.min(mn))
            rng = jnp.full((16,), jnp.max(mx)) - mn_s
            rng = jnp.where(rng <= 0.0, ones, rng)
            prm[2 * d, col] = mn_s
            prm[2 * d + 1, col] = jnp.full((16,), float(RES), jnp.float32) / rng

    def bin_process(slot, tail):
        v0 = TAIL_V0 if tail else 0
        n4 = (TAIL_V1 - TAIL_V0) // 4 if tail else NV // 4
        v1 = TAIL_V1 if tail else NV
        @pl.loop(0, 8)
        def _leg(r):
            col = pl.ds(r * 16, 16)
            mnx, sclx = prm[0, col], prm[1, col]
            mny, scly = prm[2, col], prm[3, col]
            mnz, sclz = prm[4, col], prm[5, col]
            hrow = hist1.at[pl.ds(r * F, F)]

            def binvec(xs, ys, zs, mnx=mnx, sclx=sclx, mny=mny, scly=scly,
                       mnz=mnz, sclz=sclz):
                ix = jnp.minimum((xs - mnx) * sclx, f15).astype(jnp.int32)
                iy = jnp.minimum((ys - mny) * scly, f15).astype(jnp.int32)
                iz = jnp.minimum((zs - mnz) * sclz, f15).astype(jnp.int32)
                return (ix << 8) | (iy << 4) | iz

            def quad(j4, cc, r=r, hrow=hrow, binvec=binvec):
                base = (v0 + 4 * j4) * 16
                xs = [bufs[slot, 0, r, pl.ds(base + 16 * i, 16)]
                      for i in range(4)]
                ys = [bufs[slot, 1, r, pl.ds(base + 16 * i, 16)]
                      for i in range(4)]
                zs = [bufs[slot, 2, r, pl.ds(base + 16 * i, 16)]
                      for i in range(4)]
                flats = [binvec(xs[i], ys[i], zs[i]) for i in range(4)]
                for i in range(4):
                    plsc.addupdate_scatter(hrow, [flats[i]], ones)
                return cc

            def single(v, cc, r=r, hrow=hrow, binvec=binvec):
                s = pl.ds(v * 16, 16)
                flat = binvec(bufs[slot, 0, r, s], bufs[slot, 1, r, s],
                              bufs[slot, 2, r, s])
                plsc.addupdate_scatter(hrow, [flat], ones)
                return cc

            lax.fori_loop(0, n4, quad, 0, unroll=2)
            lax.fori_loop(v0 + 4 * n4, v1, single, 0)

    run_pass(bin_process)

    for r in range(8):
        @pl.loop(0, F // 16)
        def _conv(jj, r=r):
            hist2[r, pl.ds(jj * 16, 16)] = hist1[pl.ds(r * F + jj * 16, 16)]

    pltpu.sync_copy(hist2, out_ref.at[q, pl.ds(pl.multiple_of(g * 8, 8), 8), :])


def _mm_body(p_ref, w_ref, b_ref, o_ref):
    c = (p_ref[0] + p_ref[1] + p_ref[2] + p_ref[3]) * (1.0 / float(N))
    o_ref[...] = (
        lax.dot_general(
            c, w_ref[...], (((1,), (1,)), ((), ())),
            preferred_element_type=jnp.float32,
            precision=lax.Precision.HIGHEST,
        )
        + b_ref[...]
    )


def kernel(x, W, b):
    xt = jnp.transpose(x, (2, 0, 1))
    partials = _hist_sc(xt)
    logits = pl.pallas_call(
        _mm_body,
        out_shape=jax.ShapeDtypeStruct((B, C), jnp.float32),
    )(partials, W, b.reshape(1, C))
    return logits


# speedup vs baseline: 165.6338x; 1.4606x over previous
import functools

import jax
import jax.numpy as jnp
from jax import lax
from jax.experimental import pallas as pl
from jax.experimental.pallas import tpu as pltpu
from jax.experimental.pallas import tpu_sc as plsc

RES = 16
B = 64
N = 100000
F = RES ** 3
C = 40

NC = 2
NS = 16
NQ = 4
NT = 782
CW = 23
CHC = CW * 128
NCHUNK = NT // CW
LAST = NCHUNK - 1
NV = CHC // 16
TAIL_NV = (N - LAST * CHC) // 16

_mesh = plsc.VectorSubcoreMesh(
    core_axis_name="c", subcore_axis_name="s", num_cores=NC, num_subcores=NS
)


def _idx_body(x_ref, o_ref):
    xb = x_ref[...]
    mn = jnp.min(xb, axis=2, keepdims=True)
    mx = jnp.max(xb, axis=2, keepdims=True)
    rng = mx - mn
    rng = jnp.where(rng <= 0.0, jnp.ones_like(rng), rng)
    scl = float(RES) / rng
    t = jnp.minimum((xb - mn) * scl, 15.0).astype(jnp.int32)
    o_ref[...] = (t[0] << 8) | (t[1] << 4) | t[2]


def _flat_indices(xt):
    return pl.pallas_call(
        _idx_body,
        grid=(B // 8,),
        in_specs=[pl.BlockSpec((3, 8, N), lambda g: (0, g, 0))],
        out_specs=pl.BlockSpec((8, N), lambda g: (g, 0)),
        out_shape=jax.ShapeDtypeStruct((B, N), jnp.int32),
    )(xt)


@functools.partial(
    pl.kernel,
    out_type=jax.ShapeDtypeStruct((NQ, B, F), jnp.float32),
    mesh=_mesh,
    scratch_types=[
        pltpu.VMEM((2, 8, CHC), jnp.int32),
        pltpu.VMEM((8 * F,), jnp.float32),
        pltpu.VMEM((8, F), jnp.float32),
        pltpu.SemaphoreType.DMA,
        pltpu.SemaphoreType.DMA,
    ],
    compiler_params=pltpu.CompilerParams(needs_layout_passes=False),
)
def _hist_sc(idx_ref, out_ref, bufs, hist1, hist2, semA, semB):
    cid = lax.axis_index("c")
    sid = lax.axis_index("s")
    wid = cid * NS + sid
    g = wid // NQ
    q = wid % NQ
    nq = jnp.where(q < 2, 9, 8)
    row = pl.multiple_of(g * 8, 8)

    ones = jnp.ones((16,), jnp.float32)
    zeros = jnp.zeros((16,), jnp.float32)

    @pl.loop(0, 8 * F // 16)
    def _zero(j):
        hist1[pl.ds(j * 16, 16)] = zeros

    def start_fetch(c, slot, sem):
        col = pl.multiple_of(c * CHC, 128)
        pltpu.async_copy(
            idx_ref.at[pl.ds(row, 8), pl.ds(col, CHC)], bufs.at[slot], sem
        )

    def wait_fetch(slot, sem):
        pltpu.make_async_copy(
            idx_ref.at[pl.ds(0, 8), pl.ds(0, CHC)], bufs.at[slot], sem
        ).wait()

    def process(slot, tail):
        n4 = (TAIL_NV // 4) if tail else (NV // 4)
        v1 = TAIL_NV if tail else NV

        @pl.loop(0, 8)
        def _leg(r):
            hrow = hist1.at[pl.ds(r * F, F)]

            def quad(j4, cc, hrow=hrow, r=r):
                base = j4 * 64
                flats = [bufs[slot, r, pl.ds(base + 16 * i, 16)]
                         for i in range(4)]
                for i in range(4):
                    plsc.addupdate_scatter(hrow, [flats[i]], ones)
                return cc

            def single(v, cc, hrow=hrow, r=r):
                flat = bufs[slot, r, pl.ds(v * 16, 16)]
                plsc.addupdate_scatter(hrow, [flat], ones)
                return cc

            lax.fori_loop(0, n4, quad, 0, unroll=2)
            lax.fori_loop(4 * n4, v1, single, 0)

    start_fetch(q, 0, semA)

    def step(k, carry):
        start_fetch(q + NQ * (2 * k + 1), 1, semB)
        wait_fetch(0, semA)
        process(0, False)

        @pl.when(2 * k + 2 < nq)
        def _():
            start_fetch(q + NQ * (2 * k + 2), 0, semA)

        wait_fetch(1, semB)
        process(1, False)
        return carry

    lax.fori_loop(0, nq // 2, step, 0)

    @pl.when(nq % 2 == 1)
    def _():
        wait_fetch(0, semA)

        @pl.when(q == 1)
        def _():
            process(0, True)

        @pl.when(q != 1)
        def _():
            process(0, False)

    for r in range(8):
        @pl.loop(0, F // 16)
        def _conv(jj, r=r):
            hist2[r, pl.ds(jj * 16, 16)] = hist1[pl.ds(r * F + jj * 16, 16)]

    pltpu.sync_copy(hist2, out_ref.at[q, pl.ds(row, 8), :])


def _mm_body(p_ref, w_ref, b_ref, o_ref):
    c = (p_ref[0] + p_ref[1] + p_ref[2] + p_ref[3]) * (1.0 / float(N))
    o_ref[...] = (
        lax.dot_general(
            c, w_ref[...], (((1,), (1,)), ((), ())),
            preferred_element_type=jnp.float32,
            precision=lax.Precision.HIGHEST,
        )
        + b_ref[...]
    )


def kernel(x, W, b):
    xt = jnp.transpose(x, (2, 0, 1))
    flat = _flat_indices(xt)
    partials = _hist_sc(flat)
    logits = pl.pallas_call(
        _mm_body,
        out_shape=jax.ShapeDtypeStruct((B, C), jnp.float32),
    )(partials, W, b.reshape(1, C))
    return logits


# speedup vs baseline: 165.7181x; 1.0005x over previous
import functools

import jax
import jax.numpy as jnp
from jax import lax
from jax.experimental import pallas as pl
from jax.experimental.pallas import tpu as pltpu
from jax.experimental.pallas import tpu_sc as plsc

RES = 16
B = 64
N = 100000
F = RES ** 3
C = 40

NC = 2
NS = 16
NQ = 4
NT = 782
CW = 23
CHC = CW * 128
NCHUNK = NT // CW
LAST = NCHUNK - 1
NV = CHC // 16
TAIL_NV = (N - LAST * CHC) // 16

_mesh = plsc.VectorSubcoreMesh(
    core_axis_name="c", subcore_axis_name="s", num_cores=NC, num_subcores=NS
)


def _idx_body(x_ref, o_ref):
    xb = x_ref[...]
    mn = jnp.min(xb, axis=2, keepdims=True)
    mx = jnp.max(xb, axis=2, keepdims=True)
    rng = mx - mn
    rng = jnp.where(rng <= 0.0, jnp.ones_like(rng), rng)
    scl = float(RES) / rng
    t = jnp.minimum((xb - mn) * scl, 15.0).astype(jnp.int32)
    o_ref[...] = (t[0] << 8) | (t[1] << 4) | t[2]


def _flat_indices(xt):
    return pl.pallas_call(
        _idx_body,
        grid=(B // 8,),
        in_specs=[pl.BlockSpec((3, 8, N), lambda g: (0, g, 0))],
        out_specs=pl.BlockSpec((8, N), lambda g: (g, 0)),
        out_shape=jax.ShapeDtypeStruct((B, N), jnp.int32),
    )(xt)


@functools.partial(
    pl.kernel,
    out_type=jax.ShapeDtypeStruct((NQ, B, F), jnp.float32),
    mesh=_mesh,
    scratch_types=[
        pltpu.VMEM((2, 8, CHC), jnp.int32),
        pltpu.VMEM((8 * F,), jnp.float32),
        pltpu.VMEM((8, F), jnp.float32),
        pltpu.SemaphoreType.DMA,
        pltpu.SemaphoreType.DMA,
    ],
    compiler_params=pltpu.CompilerParams(needs_layout_passes=False),
)
def _hist_sc(idx_ref, out_ref, bufs, hist1, hist2, semA, semB):
    cid = lax.axis_index("c")
    sid = lax.axis_index("s")
    wid = cid * NS + sid
    g = wid // NQ
    q = wid % NQ
    nq = jnp.where(q < 2, 9, 8)
    row = pl.multiple_of(g * 8, 8)

    ones = jnp.ones((16,), jnp.float32)
    zeros = jnp.zeros((16,), jnp.float32)

    with jax.named_scope("zero_hist"):
        @pl.loop(0, 8 * F // 16)
        def _zero(j):
            hist1[pl.ds(j * 16, 16)] = zeros

    def start_fetch(c, slot, sem):
        col = pl.multiple_of(c * CHC, 128)
        pltpu.async_copy(
            idx_ref.at[pl.ds(row, 8), pl.ds(col, CHC)], bufs.at[slot], sem
        )

    def wait_fetch(slot, sem):
        with jax.named_scope("dma_wait"):
            pltpu.make_async_copy(
                idx_ref.at[pl.ds(0, 8), pl.ds(0, CHC)], bufs.at[slot], sem
            ).wait()

    def process(slot, tail):
        n4 = (TAIL_NV // 4) if tail else (NV // 4)
        v1 = TAIL_NV if tail else NV

        @pl.loop(0, 8)
        def _leg(r):
            hrow = hist1.at[pl.ds(r * F, F)]

            def quad(j4, cc, hrow=hrow, r=r):
                base = j4 * 64
                flats = [bufs[slot, r, pl.ds(base + 16 * i, 16)]
                         for i in range(4)]
                for i in range(4):
                    plsc.addupdate_scatter(hrow, [flats[i]], ones)
                return cc

            def single(v, cc, hrow=hrow, r=r):
                flat = bufs[slot, r, pl.ds(v * 16, 16)]
                plsc.addupdate_scatter(hrow, [flat], ones)
                return cc

            with jax.named_scope("scatter"):
                lax.fori_loop(0, n4, quad, 0, unroll=2)
                lax.fori_loop(4 * n4, v1, single, 0)

    start_fetch(q, 0, semA)

    def step(k, carry):
        start_fetch(q + NQ * (2 * k + 1), 1, semB)
        wait_fetch(0, semA)
        process(0, False)

        @pl.when(2 * k + 2 < nq)
        def _():
            start_fetch(q + NQ * (2 * k + 2), 0, semA)

        wait_fetch(1, semB)
        process(1, False)
        return carry

    lax.fori_loop(0, nq // 2, step, 0)

    @pl.when(nq % 2 == 1)
    def _():
        wait_fetch(0, semA)

        @pl.when(q == 1)
        def _():
            process(0, True)

        @pl.when(q != 1)
        def _():
            process(0, False)

    with jax.named_scope("convert"):
        for r in range(8):
            @pl.loop(0, F // 16)
            def _conv(jj, r=r):
                hist2[r, pl.ds(jj * 16, 16)] = hist1[pl.ds(r * F + jj * 16, 16)]

    pltpu.sync_copy(hist2, out_ref.at[q, pl.ds(row, 8), :])


def _mm_body(p_ref, w_ref, b_ref, o_ref):
    c = (p_ref[0] + p_ref[1] + p_ref[2] + p_ref[3]) * (1.0 / float(N))
    o_ref[...] = (
        lax.dot_general(
            c, w_ref[...], (((1,), (1,)), ((), ())),
            preferred_element_type=jnp.float32,
            precision=lax.Precision.HIGHEST,
        )
        + b_ref[...]
    )


def kernel(x, W, b):
    xt = jnp.transpose(x, (2, 0, 1))
    flat = _flat_indices(xt)
    partials = _hist_sc(flat)
    logits = pl.pallas_call(
        _mm_body,
        out_shape=jax.ShapeDtypeStruct((B, C), jnp.float32),
    )(partials, W, b.reshape(1, C))
    return logits


# speedup vs baseline: 176.5933x; 1.0656x over previous
import functools

import jax
import jax.numpy as jnp
from jax import lax
from jax.experimental import pallas as pl
from jax.experimental.pallas import tpu as pltpu
from jax.experimental.pallas import tpu_sc as plsc

RES = 16
B = 64
N = 100000
F = RES ** 3
C = 40

NC = 2
NS = 16
NQ = 4
NT = 782
CW = 23
CHC = CW * 128
NCHUNK = NT // CW
LAST = NCHUNK - 1
NV = CHC // 16
TAIL_NV = (N - LAST * CHC) // 16

_mesh = plsc.VectorSubcoreMesh(
    core_axis_name="c", subcore_axis_name="s", num_cores=NC, num_subcores=NS
)


def _idx_body(x_ref, o_ref):
    xb = x_ref[...]
    mn = jnp.min(xb, axis=2, keepdims=True)
    mx = jnp.max(xb, axis=2, keepdims=True)
    rng = mx - mn
    rng = jnp.where(rng <= 0.0, jnp.ones_like(rng), rng)
    scl = float(RES) / rng
    t = jnp.minimum((xb - mn) * scl, 15.0).astype(jnp.int32)
    o_ref[...] = (t[0] << 8) | (t[1] << 4) | t[2]


def _flat_indices(xt):
    return pl.pallas_call(
        _idx_body,
        grid=(B // 8,),
        in_specs=[pl.BlockSpec((3, 8, N), lambda g: (0, g, 0))],
        out_specs=pl.BlockSpec((8, N), lambda g: (g, 0)),
        out_shape=jax.ShapeDtypeStruct((B, N), jnp.int32),
    )(xt)


@functools.partial(
    pl.kernel,
    out_type=jax.ShapeDtypeStruct((NQ, B, F), jnp.float32),
    mesh=_mesh,
    scratch_types=[
        pltpu.VMEM((2, 8, CHC), jnp.int32),
        pltpu.VMEM((8 * F,), jnp.float32),
        pltpu.VMEM((8, F), jnp.float32),
        pltpu.SemaphoreType.DMA,
        pltpu.SemaphoreType.DMA,
    ],
    compiler_params=pltpu.CompilerParams(needs_layout_passes=False),
)
def _hist_sc(idx_ref, out_ref, bufs, hist1, hist2, semA, semB):
    cid = lax.axis_index("c")
    sid = lax.axis_index("s")
    wid = cid * NS + sid
    g = wid // NQ
    q = wid % NQ
    nq = jnp.where(q < 2, 9, 8)
    row = pl.multiple_of(g * 8, 8)

    ones = jnp.ones((16,), jnp.float32)
    zeros = jnp.zeros((16,), jnp.float32)

    with jax.named_scope("zero_hist"):
        @pl.loop(0, 8 * F // 128, unroll=8)
        def _zero(j):
            for u in range(8):
                hist1[pl.ds(j * 128 + u * 16, 16)] = zeros

    def start_fetch(c, slot, sem):
        col = pl.multiple_of(c * CHC, 128)
        pltpu.async_copy(
            idx_ref.at[pl.ds(row, 8), pl.ds(col, CHC)], bufs.at[slot], sem
        )

    def wait_fetch(slot, sem):
        with jax.named_scope("dma_wait"):
            pltpu.make_async_copy(
                idx_ref.at[pl.ds(0, 8), pl.ds(0, CHC)], bufs.at[slot], sem
            ).wait()

    def process(slot, tail):
        n4 = (TAIL_NV // 4) if tail else (NV // 4)
        v1 = TAIL_NV if tail else NV

        @pl.loop(0, 8)
        def _leg(r):
            hrow = hist1.at[pl.ds(r * F, F)]

            def quad(j4, cc, hrow=hrow, r=r):
                base = j4 * 64
                flats = [bufs[slot, r, pl.ds(base + 16 * i, 16)]
                         for i in range(4)]
                for i in range(4):
                    plsc.addupdate_scatter(hrow, [flats[i]], ones)
                return cc

            def single(v, cc, hrow=hrow, r=r):
                flat = bufs[slot, r, pl.ds(v * 16, 16)]
                plsc.addupdate_scatter(hrow, [flat], ones)
                return cc

            with jax.named_scope("scatter"):
                lax.fori_loop(0, n4, quad, 0, unroll=2)
                lax.fori_loop(4 * n4, v1, single, 0)

    start_fetch(q, 0, semA)

    def step(k, carry):
        start_fetch(q + NQ * (2 * k + 1), 1, semB)
        wait_fetch(0, semA)
        process(0, False)

        @pl.when(2 * k + 2 < nq)
        def _():
            start_fetch(q + NQ * (2 * k + 2), 0, semA)

        wait_fetch(1, semB)
        process(1, False)
        return carry

    lax.fori_loop(0, nq // 2, step, 0)

    @pl.when(nq % 2 == 1)
    def _():
        wait_fetch(0, semA)

        @pl.when(q == 1)
        def _():
            process(0, True)

        @pl.when(q != 1)
        def _():
            process(0, False)

    with jax.named_scope("convert"):
        for r in range(8):
            @pl.loop(0, F // 128, unroll=4)
            def _conv(jj, r=r):
                for u in range(8):
                    s = jj * 128 + u * 16
                    hist2[r, pl.ds(s, 16)] = hist1[pl.ds(r * F + s, 16)]

    pltpu.sync_copy(hist2, out_ref.at[q, pl.ds(row, 8), :])


def _mm_body(p_ref, w_ref, b_ref, o_ref):
    c = (p_ref[0] + p_ref[1] + p_ref[2] + p_ref[3]) * (1.0 / float(N))
    o_ref[...] = (
        lax.dot_general(
            c, w_ref[...], (((1,), (1,)), ((), ())),
            preferred_element_type=jnp.float32,
            precision=lax.Precision.HIGHEST,
        )
        + b_ref[...]
    )


def kernel(x, W, b):
    xt = jnp.transpose(x, (2, 0, 1))
    flat = _flat_indices(xt)
    partials = _hist_sc(flat)
    logits = pl.pallas_call(
        _mm_body,
        out_shape=jax.ShapeDtypeStruct((B, C), jnp.float32),
    )(partials, W, b.reshape(1, C))
    return logits


# speedup vs baseline: 186.8739x; 1.0582x over previous
import functools

import jax
import jax.numpy as jnp
from jax import lax
from jax.experimental import pallas as pl
from jax.experimental.pallas import tpu as pltpu
from jax.experimental.pallas import tpu_sc as plsc

RES = 16
B = 64
N = 100000
F = RES ** 3
C = 40

NC = 2
NS = 16
NQ = 4

NPACK = 49920
NREM = N - 2 * NPACK
NT = NPACK // 128
CW = 26
CHC = CW * 128
NCHUNK = NT // CW
NV = CHC // 16

_mesh = plsc.VectorSubcoreMesh(
    core_axis_name="c", subcore_axis_name="s", num_cores=NC, num_subcores=NS
)


def _idx_body(x_ref, o_ref, o2_ref):
    xb = x_ref[...]
    mn = jnp.min(xb, axis=2, keepdims=True)
    mx = jnp.max(xb, axis=2, keepdims=True)
    rng = mx - mn
    rng = jnp.where(rng <= 0.0, jnp.ones_like(rng), rng)
    scl = float(RES) / rng
    t = jnp.minimum((xb - mn) * scl, 15.0).astype(jnp.int32)
    flat = (t[0] << 8) | (t[1] << 4) | t[2]
    o_ref[...] = (flat[:, :NPACK] << 16) | flat[:, NPACK:2 * NPACK]
    o2_ref[...] = jnp.concatenate(
        [flat[:, 2 * NPACK:], jnp.zeros((8, 256 - NREM), jnp.int32)], axis=1
    )


def _flat_indices(xt):
    return pl.pallas_call(
        _idx_body,
        grid=(B // 8,),
        in_specs=[pl.BlockSpec((3, 8, N), lambda g: (0, g, 0))],
        out_specs=[
            pl.BlockSpec((8, NPACK), lambda g: (g, 0)),
            pl.BlockSpec((8, 256), lambda g: (g, 0)),
        ],
        out_shape=[
            jax.ShapeDtypeStruct((B, NPACK), jnp.int32),
            jax.ShapeDtypeStruct((B, 256), jnp.int32),
        ],
    )(xt)


@functools.partial(
    pl.kernel,
    out_type=jax.ShapeDtypeStruct((NQ, B, F), jnp.float32),
    mesh=_mesh,
    scratch_types=[
        pltpu.VMEM((2, 8, CHC), jnp.int32),
        pltpu.VMEM((8 * F,), jnp.float32),
        pltpu.VMEM((8, F), jnp.float32),
        pltpu.VMEM((8, 256), jnp.int32),
        pltpu.SemaphoreType.DMA,
        pltpu.SemaphoreType.DMA,
    ],
    compiler_params=pltpu.CompilerParams(needs_layout_passes=False),
)
def _hist_sc(idx_ref, rem_ref, out_ref, bufs, hist1, hist2, rembuf,
             semA, semB):
    cid = lax.axis_index("c")
    sid = lax.axis_index("s")
    wid = cid * NS + sid
    g = wid // NQ
    q = wid % NQ
    nq = jnp.where(q == 3, 3, 4)
    row = pl.multiple_of(g * 8, 8)

    ones = jnp.ones((16,), jnp.float32)
    zeros = jnp.zeros((16,), jnp.float32)
    mask16 = jnp.full((16,), 0xFFFF, jnp.int32)

    with jax.named_scope("zero_hist"):
        @pl.loop(0, 8 * F // 128, unroll=8)
        def _zero(j):
            for u in range(8):
                hist1[pl.ds(j * 128 + u * 16, 16)] = zeros

    def start_fetch(c, slot, sem):
        col = pl.multiple_of(c * CHC, 128)
        pltpu.async_copy(
            idx_ref.at[pl.ds(row, 8), pl.ds(col, CHC)], bufs.at[slot], sem
        )

    def wait_fetch(slot, sem):
        with jax.named_scope("dma_wait"):
            pltpu.make_async_copy(
                idx_ref.at[pl.ds(0, 8), pl.ds(0, CHC)], bufs.at[slot], sem
            ).wait()

    def process(slot):
        @pl.loop(0, 8)
        def _leg(r):
            hrow = hist1.at[pl.ds(r * F, F)]

            def quad(j4, cc, hrow=hrow, r=r):
                base = j4 * 64
                ws = [bufs[slot, r, pl.ds(base + 16 * i, 16)]
                      for i in range(4)]
                flats = []
                for w in ws:
                    flats.append(w >> 16)
                    flats.append(w & mask16)
                for f_ in flats:
                    plsc.addupdate_scatter(hrow, [f_], ones)
                return cc

            with jax.named_scope("scatter"):
                lax.fori_loop(0, NV // 4, quad, 0, unroll=2)

    start_fetch(q, 0, semA)

    def step(k, carry):
        start_fetch(q + NQ * (2 * k + 1), 1, semB)
        wait_fetch(0, semA)
        process(0)

        @pl.when(2 * k + 2 < nq)
        def _():
            start_fetch(q + NQ * (2 * k + 2), 0, semA)

        wait_fetch(1, semB)
        process(1)
        return carry

    lax.fori_loop(0, nq // 2, step, 0)

    @pl.when(nq % 2 == 1)
    def _():
        wait_fetch(0, semA)
        process(0)

    @pl.when(q == 3)
    def _():
        pltpu.sync_copy(rem_ref.at[pl.ds(row, 8), :], rembuf)

        @pl.loop(0, 8)
        def _leg(r):
            hrow = hist1.at[pl.ds(r * F, F)]

            @pl.loop(0, NREM // 16)
            def _v(v, hrow=hrow, r=r):
                flat = rembuf[r, pl.ds(v * 16, 16)]
                plsc.addupdate_scatter(hrow, [flat], ones)

    with jax.named_scope("convert"):
        for r in range(8):
            @pl.loop(0, F // 128, unroll=4)
            def _conv(jj, r=r):
                for u in range(8):
                    s = jj * 128 + u * 16
                    hist2[r, pl.ds(s, 16)] = hist1[pl.ds(r * F + s, 16)]

    pltpu.sync_copy(hist2, out_ref.at[q, pl.ds(row, 8), :])


def _mm_body(p_ref, w_ref, b_ref, o_ref):
    c = (p_ref[0] + p_ref[1] + p_ref[2] + p_ref[3]) * (1.0 / float(N))
    o_ref[...] = (
        lax.dot_general(
            c, w_ref[...], (((1,), (1,)), ((), ())),
            preferred_element_type=jnp.float32,
            precision=lax.Precision.HIGHEST,
        )
        + b_ref[...]
    )


def kernel(x, W, b):
    xt = jnp.transpose(x, (2, 0, 1))
    packed, rem = _flat_indices(xt)
    partials = _hist_sc(packed, rem)
    logits = pl.pallas_call(
        _mm_body,
        out_shape=jax.ShapeDtypeStruct((B, C), jnp.float32),
    )(partials, W, b.reshape(1, C))
    return logits
